# Initial kernel scaffold; baseline (speedup 1.0000x reference)
import jax
import jax.numpy as jnp
from jax.experimental import pallas as pl


def kernel(x, i, j):
    raise NotImplementedError("write your pallas kernel here")


# speedup vs baseline: 3.1959x; 3.1959x over previous
import functools

import jax
import jax.numpy as jnp
from jax import lax
from jax.experimental import pallas as pl
from jax.experimental.pallas import tpu as pltpu
from jax.experimental.pallas import tpu_sc as plsc

B = 32
N = 64
D = 256
P = 4096

NC = 2
NS = 16
NW = NC * NS

C = 128
NCHUNK = P // C

_MESH = plsc.VectorSubcoreMesh(core_axis_name="c", subcore_axis_name="s")


@functools.partial(
    pl.kernel,
    mesh=_MESH,
    out_type=jax.ShapeDtypeStruct((B, P, 2 * D), jnp.float32),
    scratch_types=[
        pltpu.VMEM((NCHUNK, C), jnp.int32),
        pltpu.VMEM((C, D), jnp.float32),
        pltpu.SemaphoreType.DMA,
    ],
)
def _pair_gather(table_hbm, i_hbm, j_hbm, out_hbm, idx_v, rows_v, sem):
    w = lax.axis_index("s") * NC + lax.axis_index("c")
    base = w * N

    for half, sel_hbm in ((0, i_hbm), (1, j_hbm)):
        pltpu.sync_copy(sel_hbm, idx_v)

        def chunk_body(ci, carry, half=half):
            for t in range(C // 16):
                sl = pl.ds(t * 16, 16)
                idx_v[ci, sl] = idx_v[ci, sl] + base
            pltpu.async_copy(table_hbm.at[idx_v.at[ci]], rows_v, sem).wait()
            pltpu.sync_copy(
                rows_v,
                out_hbm.at[w, pl.ds(ci * C, C), pl.ds(half * D, D)],
            )
            return carry

        lax.fori_loop(0, NCHUNK, chunk_body, 0)


def kernel(x, i, j):
    table = x.reshape(B * N, D)
    i2 = i.reshape(NCHUNK, C)
    j2 = j.reshape(NCHUNK, C)
    return _pair_gather(table, i2, j2)


# speedup vs baseline: 3.7449x; 1.1718x over previous
import functools

import jax
import jax.numpy as jnp
from jax import lax
from jax.experimental import pallas as pl
from jax.experimental.pallas import tpu as pltpu
from jax.experimental.pallas import tpu_sc as plsc

B = 32
N = 64
D = 256
P = 4096

NC = 2
NS = 16
NW = NC * NS

C = 128
NCHUNK = P // C

_MESH = plsc.VectorSubcoreMesh(core_axis_name="c", subcore_axis_name="s")


@functools.partial(
    pl.kernel,
    mesh=_MESH,
    out_type=jax.ShapeDtypeStruct((B, P, 2 * D), jnp.float32),
    scratch_types=[
        pltpu.VMEM((NCHUNK, C), jnp.int32),
        pltpu.VMEM((C, D), jnp.float32),
        pltpu.VMEM((C, D), jnp.float32),
        pltpu.SemaphoreType.DMA,
        pltpu.SemaphoreType.DMA,
    ],
)
def _pair_gather(table_hbm, i_hbm, j_hbm, out_hbm, idx_v, rows0, rows1,
                 gsem0, gsem1):
    w = lax.axis_index("s") * NC + lax.axis_index("c")
    base = w * N
    bufs = ((rows0, gsem0), (rows1, gsem1))

    for half, sel_hbm in ((0, i_hbm), (1, j_hbm)):
        pltpu.sync_copy(sel_hbm, idx_v)

        def prep_body(ci, carry):
            for t in range(C // 16):
                sl = pl.ds(t * 16, 16)
                idx_v[ci, sl] = idx_v[ci, sl] + base
            return carry

        lax.fori_loop(0, NCHUNK, prep_body, 0)

        for b, (rows, gsem) in enumerate(bufs):
            pltpu.async_copy(table_hbm.at[idx_v.at[b]], rows, gsem)

        def chunk_body(g, carry, half=half):
            for b, (rows, gsem) in enumerate(bufs):
                ci = 2 * g + b
                pltpu.make_async_copy(
                    table_hbm.at[idx_v.at[ci]], rows, gsem).wait()
                pltpu.sync_copy(
                    rows,
                    out_hbm.at[w, pl.ds(ci * C, C), pl.ds(half * D, D)],
                )
                nci = ci + 2

                @pl.when(nci < NCHUNK)
                def _():
                    pltpu.async_copy(table_hbm.at[idx_v.at[nci]], rows, gsem)
            return carry

        lax.fori_loop(0, NCHUNK // 2, chunk_body, 0)


def kernel(x, i, j):
    table = x.reshape(B * N, D)
    i2 = i.reshape(NCHUNK, C)
    j2 = j.reshape(NCHUNK, C)
    return _pair_gather(table, i2, j2)
